# baseline (speedup 1.0000x reference)
import jax
import jax.numpy as jnp
from jax.experimental import pallas as pl
from jax.experimental.pallas import tpu as pltpu


def _fused_patch_conv_kernel(w_ref, x_ref, b_ref, o_ref):
    x = x_ref[...]
    p = (
        x.reshape(3, 14, 16, 14, 16)
        .transpose(0, 2, 4, 1, 3)
        .reshape(768, 196)
    )
    acc = jnp.dot(w_ref[...], p, preferred_element_type=jnp.float32)
    acc = jnp.maximum(acc + b_ref[...], 0.0)
    o_ref[...] = acc.astype(o_ref.dtype)


def kernel(x, weight, bias):
    N, Cin, H, W = x.shape
    Cout = weight.shape[0]
    k = 16
    Ho, Wo = H // k, W // k
    M = Ho * Wo
    K = Cin * k * k

    w_mat = weight.reshape(Cout, K)
    b_col = bias.reshape(Cout, 1)

    out = pl.pallas_call(
        _fused_patch_conv_kernel,
        out_shape=jax.ShapeDtypeStruct((N, Cout, M), x.dtype),
        grid_spec=pl.GridSpec(
            grid=(N,),
            in_specs=[
                pl.BlockSpec((Cout, K), lambda n: (0, 0)),
                pl.BlockSpec((None, Cin, H, W), lambda n: (n, 0, 0, 0)),
                pl.BlockSpec((Cout, 1), lambda n: (0, 0)),
            ],
            out_specs=pl.BlockSpec((None, Cout, M), lambda n: (n, 0, 0)),
        ),
        compiler_params=pltpu.CompilerParams(
            dimension_semantics=("parallel",)),
    )(w_mat, x, b_col)

    return out.reshape(N, Cout, Ho, Wo)


# speedup vs baseline: 2.2132x; 2.2132x over previous
import jax
import jax.numpy as jnp
from jax.experimental import pallas as pl
from jax.experimental.pallas import tpu as pltpu


def _matmul_bias_relu_kernel(w_ref, p_ref, b_ref, o_ref):
    acc = jnp.dot(w_ref[...], p_ref[...],
                  preferred_element_type=jnp.float32)
    acc = jnp.maximum(acc + b_ref[...], 0.0)
    o_ref[...] = acc.astype(o_ref.dtype)


def kernel(x, weight, bias):
    N, Cin, H, W = x.shape
    Cout = weight.shape[0]
    k = 16
    Ho, Wo = H // k, W // k
    M = Ho * Wo
    K = Cin * k * k

    patches = (
        x.reshape(N, Cin, Ho, k, Wo, k)
        .transpose(0, 1, 3, 5, 2, 4)
        .reshape(N, K, M)
        .astype(jnp.bfloat16)
    )
    w_mat = weight.reshape(Cout, K).astype(jnp.bfloat16)
    b_col = bias.reshape(Cout, 1)

    out = pl.pallas_call(
        _matmul_bias_relu_kernel,
        out_shape=jax.ShapeDtypeStruct((N, Cout, M), x.dtype),
        grid_spec=pl.GridSpec(
            grid=(N,),
            in_specs=[
                pl.BlockSpec((Cout, K), lambda n: (0, 0)),
                pl.BlockSpec((None, K, M), lambda n: (n, 0, 0)),
                pl.BlockSpec((Cout, 1), lambda n: (0, 0)),
            ],
            out_specs=pl.BlockSpec((None, Cout, M), lambda n: (n, 0, 0)),
        ),
        compiler_params=pltpu.CompilerParams(
            dimension_semantics=("parallel",)),
    )(w_mat, patches, b_col)

    return out.reshape(N, Cout, Ho, Wo)


# speedup vs baseline: 2.6953x; 1.2178x over previous
import jax
import jax.numpy as jnp
from jax.experimental import pallas as pl
from jax.experimental.pallas import tpu as pltpu


_IMGS_PER_STEP = 8


def _matmul_bias_relu_kernel(w_ref, p_ref, b_ref, o_ref):
    w = w_ref[...]
    b = b_ref[...]
    for i in range(_IMGS_PER_STEP):
        acc = jnp.dot(w, p_ref[i], preferred_element_type=jnp.float32)
        o_ref[i] = jnp.maximum(acc + b, 0.0).astype(o_ref.dtype)


def kernel(x, weight, bias):
    N, Cin, H, W = x.shape
    Cout = weight.shape[0]
    k = 16
    Ho, Wo = H // k, W // k
    M = Ho * Wo
    K = Cin * k * k

    patches = (
        x.reshape(N, Cin, Ho, k, Wo, k)
        .transpose(0, 1, 3, 5, 2, 4)
        .reshape(N, K, M)
        .astype(jnp.bfloat16)
    )
    w_mat = weight.reshape(Cout, K).astype(jnp.bfloat16)
    b_col = bias.reshape(Cout, 1)

    out = pl.pallas_call(
        _matmul_bias_relu_kernel,
        out_shape=jax.ShapeDtypeStruct((N, Cout, M), x.dtype),
        grid_spec=pl.GridSpec(
            grid=(N // _IMGS_PER_STEP,),
            in_specs=[
                pl.BlockSpec((Cout, K), lambda n: (0, 0)),
                pl.BlockSpec((_IMGS_PER_STEP, K, M), lambda n: (n, 0, 0)),
                pl.BlockSpec((Cout, 1), lambda n: (0, 0)),
            ],
            out_specs=pl.BlockSpec((_IMGS_PER_STEP, Cout, M),
                                   lambda n: (n, 0, 0)),
        ),
        compiler_params=pltpu.CompilerParams(
            dimension_semantics=("parallel",)),
    )(w_mat, patches, b_col)

    return out.reshape(N, Cout, Ho, Wo)
